# Initial kernel scaffold; baseline (speedup 1.0000x reference)
import jax
import jax.numpy as jnp
from jax.experimental import pallas as pl


def kernel(gh_label, gah_label, p_gh, p_gah, mask):
    raise NotImplementedError("write your pallas kernel here")


# speedup vs baseline: 86.3685x; 86.3685x over previous
import functools

import jax
import jax.numpy as jnp
from jax import lax
from jax.experimental import pallas as pl
from jax.experimental.pallas import tpu as pltpu
from jax.experimental.pallas import tpu_sc as plsc

_NC = 2
_NS = 16
_NW = _NC * _NS
_LANES = 16

_IMGS = 12
_M = 384 * 384
_CHUNK = _M // _NW
_NV = _CHUNK // _LANES

@functools.cache
def _get_stage1():
    mesh = plsc.VectorSubcoreMesh(
        core_axis_name="c", subcore_axis_name="s",
        num_cores=_NC, num_subcores=_NS,
    )
    return functools.partial(
        pl.kernel,
        out_type=jax.ShapeDtypeStruct((_NW, _IMGS * _LANES), jnp.float32),
        mesh=mesh,
        scratch_types=[
            pltpu.VMEM((_CHUNK,), jnp.float32),
            pltpu.VMEM((_CHUNK,), jnp.float32),
            pltpu.VMEM((_CHUNK,), jnp.float32),
            pltpu.VMEM((_CHUNK,), jnp.float32),
            pltpu.VMEM((_CHUNK,), jnp.float32),
            pltpu.VMEM((_IMGS * _LANES,), jnp.float32),
        ],
        compiler_params=pltpu.CompilerParams(needs_layout_passes=False),
    )(_stage1_body)


def _stage1_body(gh_hbm, gah_hbm, pgh_hbm, pga_hbm, msk_hbm, out_hbm,
                 gv, av, qv, bv, mv, resv):
    wid = lax.axis_index("s") * _NC + lax.axis_index("c")
    lane = lax.iota(jnp.int32, _LANES)
    zero = jnp.zeros((_LANES,), jnp.float32)
    for i in range(_IMGS):
        base = i * _M + wid * _CHUNK
        pltpu.sync_copy(gh_hbm.at[pl.ds(base, _CHUNK)], gv)
        pltpu.sync_copy(gah_hbm.at[pl.ds(base, _CHUNK)], av)
        pltpu.sync_copy(pgh_hbm.at[pl.ds(base, _CHUNK)], qv)
        pltpu.sync_copy(pga_hbm.at[pl.ds(base, _CHUNK)], bv)
        pltpu.sync_copy(msk_hbm.at[pl.ds(base, _CHUNK)], mv)

        def body(j, carry):
            pg, ng, spg, sng, pa, na, spa, sna = carry
            sl = pl.ds(j * _LANES, _LANES)
            lg = gv[sl]
            la = av[sl]
            mk = mv[sl]
            dg = qv[sl] - lg
            da = bv[sl] - la
            plg = dg * dg * mk
            pla = da * da * mk
            pos_g = lg > 0.1
            neg_g = lg < 0.1
            pos_a = la > 0.1
            neg_a = la < 0.1
            pg = pg + jnp.where(pos_g, 1.0, 0.0)
            ng = ng + jnp.where(neg_g, 1.0, 0.0)
            spg = spg + jnp.where(pos_g, plg, 0.0)
            sng = sng + jnp.where(neg_g, plg, 0.0)
            pa = pa + jnp.where(pos_a, 1.0, 0.0)
            na = na + jnp.where(neg_a, 1.0, 0.0)
            spa = spa + jnp.where(pos_a, pla, 0.0)
            sna = sna + jnp.where(neg_a, pla, 0.0)
            return pg, ng, spg, sng, pa, na, spa, sna

        acc = lax.fori_loop(0, _NV, body, (zero,) * 8)
        res = zero
        for q in range(8):
            res = jnp.where(lane == q, jnp.sum(acc[q]), res)
        resv[pl.ds(i * _LANES, _LANES)] = res
    pltpu.sync_copy(resv, out_hbm.at[wid])


def _sel_body(k_ref, lab_ref, pred_ref, msk_ref, tk_ref, t5_ref):
    r = pl.program_id(0)
    lab = lab_ref[0]
    d = pred_ref[0] - lab
    plv = d * d * msk_ref[0]
    bits = lax.bitcast_convert_type(plv, jnp.int32)
    negbits = jnp.where(lab < 0.1, bits, -1)
    kk = k_ref[r]

    def srch(b, carry):
        t1, t2 = carry
        bit = jnp.left_shift(jnp.int32(1), 30 - b)
        tr1 = t1 | bit
        tr2 = t2 | bit
        c1 = jnp.sum((negbits >= tr1).astype(jnp.int32))
        c2 = jnp.sum((bits >= tr2).astype(jnp.int32))
        t1 = jnp.where(c1 >= kk, tr1, t1)
        t2 = jnp.where(c2 >= 500, tr2, t2)
        return t1, t2

    t1, t2 = lax.fori_loop(0, 31, srch, (jnp.int32(0), jnp.int32(0)))

    thr1 = lax.bitcast_convert_type(jnp.full((1, 1, 128), t1, jnp.int32),
                                    jnp.float32)
    thr2 = lax.bitcast_convert_type(jnp.full((1, 1, 128), t2, jnp.int32),
                                    jnp.float32)
    gt1 = negbits > t1
    gt2 = bits > t2
    c1 = jnp.sum(gt1.astype(jnp.float32))
    c2 = jnp.sum(gt2.astype(jnp.float32))
    s1 = jnp.sum(jnp.where(gt1, plv, 0.0))
    s2 = jnp.sum(jnp.where(gt2, plv, 0.0))
    tk_ref[...] = s1 + (kk.astype(jnp.float32) - c1) * thr1
    t5_ref[...] = s2 + (500.0 - c2) * thr2


def _selection(kk, labs, preds, mask):
    tk, t5 = pl.pallas_call(
        _sel_body,
        grid=(2 * _IMGS,),
        in_specs=[
            pl.BlockSpec(memory_space=pltpu.SMEM),
            pl.BlockSpec((1, 384, 384), lambda r: (r, 0, 0)),
            pl.BlockSpec((1, 384, 384), lambda r: (r, 0, 0)),
            pl.BlockSpec((1, 384, 384), lambda r: (r % _IMGS, 0, 0)),
        ],
        out_specs=[
            pl.BlockSpec((1, 1, 128), lambda r: (r, 0, 0)),
            pl.BlockSpec((1, 1, 128), lambda r: (r, 0, 0)),
        ],
        out_shape=[
            jax.ShapeDtypeStruct((2 * _IMGS, 1, 128), jnp.float32),
            jax.ShapeDtypeStruct((2 * _IMGS, 1, 128), jnp.float32),
        ],
    )(kk, labs, preds, mask)
    return tk[:, 0, 0], t5[:, 0, 0]


def kernel(gh_label, gah_label, p_gh, p_gah, mask):
    parts = _get_stage1()(
        gh_label.reshape(-1),
        gah_label.reshape(-1),
        p_gh.reshape(-1),
        p_gah.reshape(-1),
        mask.reshape(-1),
    )
    tot = parts.sum(axis=0).reshape(_IMGS, _LANES)
    p = jnp.stack([tot[:, 0], tot[:, 4]])
    n = jnp.stack([tot[:, 1], tot[:, 5]])
    sp = jnp.stack([tot[:, 2], tot[:, 6]])
    sn = jnp.stack([tot[:, 3], tot[:, 7]])
    k = 3.0 * p

    need_sel = jnp.any((p == 0.0) | (n >= k))

    def sel_true(_):
        labs = jnp.concatenate([gh_label, gah_label], axis=0)
        preds = jnp.concatenate([p_gh, p_gah], axis=0)
        kints = k.reshape(-1).astype(jnp.int32)
        return _selection(kints, labs, preds, mask)

    def sel_false(_):
        z = jnp.zeros((2 * _IMGS,), jnp.float32)
        return z, z

    tk, t5 = lax.cond(need_sel, sel_true, sel_false, 0)
    tk = tk.reshape(2, _IMGS)
    t5 = t5.reshape(2, _IMGS)

    posi = sp / p
    nega = jnp.where(n < k, sn / n, tk / k)
    row = jnp.where(p != 0.0, posi + nega, t5 / 500.0)
    return jnp.sum(row) / 12.0


# speedup vs baseline: 131.0480x; 1.5173x over previous
import functools

import jax
import jax.numpy as jnp
from jax import lax
from jax.experimental import pallas as pl
from jax.experimental.pallas import tpu as pltpu
from jax.experimental.pallas import tpu_sc as plsc

_NC = 2
_NS = 16
_NW = _NC * _NS
_LANES = 16

_IMGS = 12
_M = 384 * 384
_CHUNK = _M // _NW
_NV = _CHUNK // _LANES

@functools.cache
def _get_stage1():
    mesh = plsc.VectorSubcoreMesh(
        core_axis_name="c", subcore_axis_name="s",
        num_cores=_NC, num_subcores=_NS,
    )
    return functools.partial(
        pl.kernel,
        out_type=jax.ShapeDtypeStruct((_NW, _IMGS * _LANES), jnp.float32),
        mesh=mesh,
        scratch_types=[pltpu.VMEM((_CHUNK,), jnp.float32)] * 10 + [
            pltpu.VMEM((_IMGS * _LANES,), jnp.float32),
            pltpu.SemaphoreType.DMA,
            pltpu.SemaphoreType.DMA,
        ],
        compiler_params=pltpu.CompilerParams(needs_layout_passes=False),
    )(_stage1_body)


_UNROLL = 4


def _stage1_body(gh_hbm, gah_hbm, pgh_hbm, pga_hbm, msk_hbm, out_hbm,
                 g0, a0, q0, b0, m0, g1, a1, q1, b1, m1, resv, sem0, sem1):
    wid = lax.axis_index("s") * _NC + lax.axis_index("c")
    lane = lax.iota(jnp.int32, _LANES)
    zero = jnp.zeros((_LANES,), jnp.float32)
    hbm = (gh_hbm, gah_hbm, pgh_hbm, pga_hbm, msk_hbm)
    bufs = ((g0, a0, q0, b0, m0), (g1, a1, q1, b1, m1))
    sems = (sem0, sem1)

    def start_img(i):
        sem = sems[i % 2]
        hs = []
        for src, dst in zip(hbm, bufs[i % 2]):
            h = pltpu.make_async_copy(
                src.at[pl.ds(i * _M + wid * _CHUNK, _CHUNK)], dst, sem)
            h.start()
            hs.append(h)
        return hs

    pend = start_img(0)
    for i in range(_IMGS):
        for h in pend:
            h.wait()
        if i + 1 < _IMGS:
            pend = start_img(i + 1)
        gv, av, qv, bv, mv = bufs[i % 2]

        def body(j, carry):
            pg, ng, spg, sng, pa, na, spa, sna = carry
            for u in range(_UNROLL):
                sl = pl.ds((j * _UNROLL + u) * _LANES, _LANES)
                lg = gv[sl]
                la = av[sl]
                mk = mv[sl]
                dg = qv[sl] - lg
                da = bv[sl] - la
                plg = dg * dg * mk
                pla = da * da * mk
                pos_g = lg > 0.1
                neg_g = lg < 0.1
                pos_a = la > 0.1
                neg_a = la < 0.1
                pg = pg + jnp.where(pos_g, 1.0, 0.0)
                ng = ng + jnp.where(neg_g, 1.0, 0.0)
                spg = spg + jnp.where(pos_g, plg, 0.0)
                sng = sng + jnp.where(neg_g, plg, 0.0)
                pa = pa + jnp.where(pos_a, 1.0, 0.0)
                na = na + jnp.where(neg_a, 1.0, 0.0)
                spa = spa + jnp.where(pos_a, pla, 0.0)
                sna = sna + jnp.where(neg_a, pla, 0.0)
            return pg, ng, spg, sng, pa, na, spa, sna

        acc = lax.fori_loop(0, _NV // _UNROLL, body, (zero,) * 8)
        res = zero
        for q in range(8):
            res = jnp.where(lane == q, jnp.sum(acc[q]), res)
        resv[pl.ds(i * _LANES, _LANES)] = res
    pltpu.sync_copy(resv, out_hbm.at[wid])


def _sel_body(k_ref, lab_ref, pred_ref, msk_ref, tk_ref, t5_ref):
    r = pl.program_id(0)
    lab = lab_ref[0]
    d = pred_ref[0] - lab
    plv = d * d * msk_ref[0]
    bits = lax.bitcast_convert_type(plv, jnp.int32)
    negbits = jnp.where(lab < 0.1, bits, -1)
    kk = k_ref[r]

    def srch(b, carry):
        t1, t2 = carry
        bit = jnp.left_shift(jnp.int32(1), 30 - b)
        tr1 = t1 | bit
        tr2 = t2 | bit
        c1 = jnp.sum((negbits >= tr1).astype(jnp.int32))
        c2 = jnp.sum((bits >= tr2).astype(jnp.int32))
        t1 = jnp.where(c1 >= kk, tr1, t1)
        t2 = jnp.where(c2 >= 500, tr2, t2)
        return t1, t2

    t1, t2 = lax.fori_loop(0, 31, srch, (jnp.int32(0), jnp.int32(0)))

    thr1 = lax.bitcast_convert_type(jnp.full((1, 1, 128), t1, jnp.int32),
                                    jnp.float32)
    thr2 = lax.bitcast_convert_type(jnp.full((1, 1, 128), t2, jnp.int32),
                                    jnp.float32)
    gt1 = negbits > t1
    gt2 = bits > t2
    c1 = jnp.sum(gt1.astype(jnp.float32))
    c2 = jnp.sum(gt2.astype(jnp.float32))
    s1 = jnp.sum(jnp.where(gt1, plv, 0.0))
    s2 = jnp.sum(jnp.where(gt2, plv, 0.0))
    tk_ref[...] = s1 + (kk.astype(jnp.float32) - c1) * thr1
    t5_ref[...] = s2 + (500.0 - c2) * thr2


def _selection(kk, labs, preds, mask):
    tk, t5 = pl.pallas_call(
        _sel_body,
        grid=(2 * _IMGS,),
        in_specs=[
            pl.BlockSpec(memory_space=pltpu.SMEM),
            pl.BlockSpec((1, 384, 384), lambda r: (r, 0, 0)),
            pl.BlockSpec((1, 384, 384), lambda r: (r, 0, 0)),
            pl.BlockSpec((1, 384, 384), lambda r: (r % _IMGS, 0, 0)),
        ],
        out_specs=[
            pl.BlockSpec((1, 1, 128), lambda r: (r, 0, 0)),
            pl.BlockSpec((1, 1, 128), lambda r: (r, 0, 0)),
        ],
        out_shape=[
            jax.ShapeDtypeStruct((2 * _IMGS, 1, 128), jnp.float32),
            jax.ShapeDtypeStruct((2 * _IMGS, 1, 128), jnp.float32),
        ],
    )(kk, labs, preds, mask)
    return tk[:, 0, 0], t5[:, 0, 0]


def kernel(gh_label, gah_label, p_gh, p_gah, mask):
    parts = _get_stage1()(
        gh_label.reshape(-1),
        gah_label.reshape(-1),
        p_gh.reshape(-1),
        p_gah.reshape(-1),
        mask.reshape(-1),
    )
    tot = parts.sum(axis=0).reshape(_IMGS, _LANES)
    p = jnp.stack([tot[:, 0], tot[:, 4]])
    n = jnp.stack([tot[:, 1], tot[:, 5]])
    sp = jnp.stack([tot[:, 2], tot[:, 6]])
    sn = jnp.stack([tot[:, 3], tot[:, 7]])
    k = 3.0 * p

    need_sel = jnp.any((p == 0.0) | (n >= k))

    def sel_true(_):
        labs = jnp.concatenate([gh_label, gah_label], axis=0)
        preds = jnp.concatenate([p_gh, p_gah], axis=0)
        kints = k.reshape(-1).astype(jnp.int32)
        return _selection(kints, labs, preds, mask)

    def sel_false(_):
        z = jnp.zeros((2 * _IMGS,), jnp.float32)
        return z, z

    tk, t5 = lax.cond(need_sel, sel_true, sel_false, 0)
    tk = tk.reshape(2, _IMGS)
    t5 = t5.reshape(2, _IMGS)

    posi = sp / p
    nega = jnp.where(n < k, sn / n, tk / k)
    row = jnp.where(p != 0.0, posi + nega, t5 / 500.0)
    return jnp.sum(row) / 12.0
